# Initial kernel scaffold; baseline (speedup 1.0000x reference)
import jax
import jax.numpy as jnp
from jax import lax
from jax.experimental import pallas as pl
from jax.experimental.pallas import tpu as pltpu
from jax.experimental.pallas import tpu_sc as plsc

EMB_DIM = 64
NUM_NEG = 5
NC = 2
NS = 16
NW = NC * NS
LANES = 16
CHUNK = 64
GROUPS = CHUNK // LANES

_LN2 = 0.6931471805599453


def _softplus(x):
    t = jnp.exp(x)
    y = 1.0 + t
    b = lax.bitcast_convert_type(y, jnp.int32)
    e = (b >> 23) - 127
    m = lax.bitcast_convert_type((b & 0x007FFFFF) | 0x3F800000, jnp.float32)
    z = (m - 1.0) / (m + 1.0)
    z2 = z * z
    p = z * (2.0 + z2 * (0.66666667 + z2 * (0.4 + z2 * (0.28571429 + z2 * 0.22222222))))
    return e.astype(jnp.float32) * _LN2 + p


def _body(pos_u, pos_v, neg_f, u_emb, v_emb, out,
          pu_idx, pv_idx, ng_idx,
          u_buf0, u_buf1, v_buf0, v_buf1, n_buf0, n_buf1,
          acc_buf, sem0, sem1):
    wid = lax.axis_index("s") * NC + lax.axis_index("c")
    per_tile = 512
    base = wid * per_tile
    n_chunks = per_tile // CHUNK

    pltpu.sync_copy(pos_u.at[pl.ds(base, per_tile)], pu_idx)
    pltpu.sync_copy(pos_v.at[pl.ds(base, per_tile)], pv_idx)
    pltpu.sync_copy(neg_f.at[pl.ds(base * NUM_NEG, per_tile * NUM_NEG)], ng_idx)

    u_bufs = (u_buf0, u_buf1)
    v_bufs = (v_buf0, v_buf1)
    n_bufs = (n_buf0, n_buf1)
    sems = (sem0, sem1)

    def fire(c, slot):
        nb = CHUNK * NUM_NEG
        return [
            pltpu.async_copy(u_emb.at[pu_idx.at[pl.ds(c * CHUNK, CHUNK)]],
                             u_bufs[slot], sems[slot]),
            pltpu.async_copy(v_emb.at[pv_idx.at[pl.ds(c * CHUNK, CHUNK)]],
                             v_bufs[slot], sems[slot]),
            pltpu.async_copy(v_emb.at[ng_idx.at[pl.ds(c * nb, 128)]],
                             n_bufs[slot].at[pl.ds(0, 128)], sems[slot]),
            pltpu.async_copy(v_emb.at[ng_idx.at[pl.ds(c * nb + 128, 128)]],
                             n_bufs[slot].at[pl.ds(128, 128)], sems[slot]),
            pltpu.async_copy(v_emb.at[ng_idx.at[pl.ds(c * nb + 256, 64)]],
                             n_bufs[slot].at[pl.ds(256, 64)], sems[slot]),
        ]

    lane_iota = lax.iota(jnp.int32, LANES)
    acc = jnp.zeros((LANES,), jnp.float32)

    def compute(slot, acc):
        ub, vb, nb = u_bufs[slot], v_bufs[slot], n_bufs[slot]

        def group_step(g, acc):
            item = lane_iota + g * LANES
            nrow0 = item * NUM_NEG

            def d_step(d, carry):
                s, n0, n1, n2, n3, n4 = carry
                dv = jnp.full((LANES,), d, jnp.int32)
                uc = plsc.load_gather(ub, [item, dv])
                vc = plsc.load_gather(vb, [item, dv])
                s = s + uc * vc
                n0 = n0 + plsc.load_gather(nb, [nrow0, dv]) * uc
                n1 = n1 + plsc.load_gather(nb, [nrow0 + 1, dv]) * uc
                n2 = n2 + plsc.load_gather(nb, [nrow0 + 2, dv]) * uc
                n3 = n3 + plsc.load_gather(nb, [nrow0 + 3, dv]) * uc
                n4 = n4 + plsc.load_gather(nb, [nrow0 + 4, dv]) * uc
                return s, n0, n1, n2, n3, n4

            z = jnp.zeros((LANES,), jnp.float32)
            s, n0, n1, n2, n3, n4 = lax.fori_loop(
                0, EMB_DIM, d_step, (z, z, z, z, z, z))
            acc = acc + _softplus(-jnp.clip(s, -10.0, 10.0))
            for nk in (n0, n1, n2, n3, n4):
                acc = acc + _softplus(jnp.clip(nk, -10.0, 10.0))
            return acc

        return lax.fori_loop(0, GROUPS, group_step, acc)

    pending = fire(0, 0)
    for c in range(n_chunks):
        nxt = fire(c + 1, (c + 1) % 2) if c + 1 < n_chunks else None
        for cp in pending:
            cp.wait()
        acc = compute(c % 2, acc)
        pending = nxt

    acc_buf[...] = acc
    pltpu.sync_copy(acc_buf, out.at[wid])


@jax.jit
def _sc_skipgram(pos_u, pos_v, neg_f, u_emb, v_emb):
    mesh = plsc.VectorSubcoreMesh(core_axis_name="c", subcore_axis_name="s")
    kcall = pl.kernel(
        _body,
        out_type=jax.ShapeDtypeStruct((NW, LANES), jnp.float32),
        mesh=mesh,
        scratch_types=[
            pltpu.VMEM((512,), jnp.int32),
            pltpu.VMEM((512,), jnp.int32),
            pltpu.VMEM((512 * NUM_NEG,), jnp.int32),
            pltpu.VMEM((CHUNK, EMB_DIM), jnp.float32),
            pltpu.VMEM((CHUNK, EMB_DIM), jnp.float32),
            pltpu.VMEM((CHUNK, EMB_DIM), jnp.float32),
            pltpu.VMEM((CHUNK, EMB_DIM), jnp.float32),
            pltpu.VMEM((CHUNK * NUM_NEG, EMB_DIM), jnp.float32),
            pltpu.VMEM((CHUNK * NUM_NEG, EMB_DIM), jnp.float32),
            pltpu.VMEM((LANES,), jnp.float32),
            pltpu.SemaphoreType.DMA,
            pltpu.SemaphoreType.DMA,
        ],
    )
    return kcall(pos_u, pos_v, neg_f, u_emb, v_emb)


def kernel(pos_u, pos_v, neg_v, u_emb, v_emb):
    batch = pos_u.shape[0]
    neg_f = neg_v.astype(jnp.int32).reshape(-1)
    partials = _sc_skipgram(pos_u.astype(jnp.int32), pos_v.astype(jnp.int32),
                            neg_f, u_emb, v_emb)
    return jnp.sum(partials) * (1.0 / batch)


# speedup vs baseline: 1.5906x; 1.5906x over previous
import jax
import jax.numpy as jnp
from jax import lax
from jax.experimental import pallas as pl
from jax.experimental.pallas import tpu as pltpu
from jax.experimental.pallas import tpu_sc as plsc

EMB_DIM = 64
NUM_NEG = 5
NC = 2
NS = 16
NW = NC * NS
LANES = 16
PER_TILE = 512
CHUNK = 64
GROUPS = CHUNK // LANES
NROWS = CHUNK * NUM_NEG

_LN2 = 0.6931471805599453


def _softplus(x):
    t = jnp.exp(x)
    y = 1.0 + t
    b = lax.bitcast_convert_type(y, jnp.int32)
    e = (b >> 23) - 127
    m = lax.bitcast_convert_type((b & 0x007FFFFF) | 0x3F800000, jnp.float32)
    z = (m - 1.0) / (m + 1.0)
    z2 = z * z
    p = z * (2.0 + z2 * (0.66666667 + z2 * (0.4 + z2 * (0.28571429 + z2 * 0.22222222))))
    return e.astype(jnp.float32) * _LN2 + p


def _body(pos_u, pos_v, neg_f, u_emb2, v_emb2, out,
          pu_idx, pv_idx, ng_idx, pu_half, pv_half, ng_half,
          u_buf0, u_buf1, v_buf0, v_buf1, n_buf0, n_buf1,
          acc_buf, sem0, sem1):
    wid = lax.axis_index("s") * NC + lax.axis_index("c")
    base = wid * PER_TILE
    n_chunks = PER_TILE // CHUNK

    pltpu.sync_copy(pos_u.at[pl.ds(base, PER_TILE)], pu_idx)
    pltpu.sync_copy(pos_v.at[pl.ds(base, PER_TILE)], pv_idx)
    pltpu.sync_copy(neg_f.at[pl.ds(base * NUM_NEG, PER_TILE * NUM_NEG)], ng_idx)

    def halve(i, _):
        pu_half[pl.ds(i * LANES, LANES)] = pu_idx[pl.ds(i * LANES, LANES)] >> 1
        pv_half[pl.ds(i * LANES, LANES)] = pv_idx[pl.ds(i * LANES, LANES)] >> 1
        return 0
    lax.fori_loop(0, PER_TILE // LANES, halve, 0)

    def halve_n(i, _):
        ng_half[pl.ds(i * LANES, LANES)] = ng_idx[pl.ds(i * LANES, LANES)] >> 1
        return 0
    lax.fori_loop(0, PER_TILE * NUM_NEG // LANES, halve_n, 0)

    u_bufs = (u_buf0, u_buf1)
    v_bufs = (v_buf0, v_buf1)
    n_bufs = (n_buf0, n_buf1)
    sems = (sem0, sem1)

    def fire(c, slot):
        return [
            pltpu.async_copy(u_emb2.at[pu_half.at[pl.ds(c * CHUNK, CHUNK)]],
                             u_bufs[slot], sems[slot]),
            pltpu.async_copy(v_emb2.at[pv_half.at[pl.ds(c * CHUNK, CHUNK)]],
                             v_bufs[slot], sems[slot]),
            pltpu.async_copy(v_emb2.at[ng_half.at[pl.ds(c * NROWS, 128)]],
                             n_bufs[slot].at[pl.ds(0, 128)], sems[slot]),
            pltpu.async_copy(v_emb2.at[ng_half.at[pl.ds(c * NROWS + 128, 128)]],
                             n_bufs[slot].at[pl.ds(128, 128)], sems[slot]),
            pltpu.async_copy(v_emb2.at[ng_half.at[pl.ds(c * NROWS + 256, 64)]],
                             n_bufs[slot].at[pl.ds(256, 64)], sems[slot]),
        ]

    lane_iota = lax.iota(jnp.int32, LANES)
    acc0 = jnp.zeros((LANES,), jnp.float32)

    def compute(c, slot, acc):
        ub, vb, nb = u_bufs[slot], v_bufs[slot], n_bufs[slot]

        def group_step(g, acc):
            item = lane_iota + g * LANES
            nrow0 = item * NUM_NEG
            paru = (pu_idx[pl.ds(c * CHUNK + g * LANES, LANES)] & 1) * EMB_DIM
            parv = (pv_idx[pl.ds(c * CHUNK + g * LANES, LANES)] & 1) * EMB_DIM
            parn = [
                (plsc.load_gather(ng_idx, [c * NROWS + nrow0 + n]) & 1) * EMB_DIM
                for n in range(NUM_NEG)
            ]

            def d_step(d, carry):
                s, n0, n1, n2, n3, n4 = carry
                dv = jnp.full((LANES,), d, jnp.int32)
                uc = plsc.load_gather(ub, [item, dv + paru])
                vc = plsc.load_gather(vb, [item, dv + parv])
                s = s + uc * vc
                n0 = n0 + plsc.load_gather(nb, [nrow0, dv + parn[0]]) * uc
                n1 = n1 + plsc.load_gather(nb, [nrow0 + 1, dv + parn[1]]) * uc
                n2 = n2 + plsc.load_gather(nb, [nrow0 + 2, dv + parn[2]]) * uc
                n3 = n3 + plsc.load_gather(nb, [nrow0 + 3, dv + parn[3]]) * uc
                n4 = n4 + plsc.load_gather(nb, [nrow0 + 4, dv + parn[4]]) * uc
                return s, n0, n1, n2, n3, n4

            z = jnp.zeros((LANES,), jnp.float32)
            s, n0, n1, n2, n3, n4 = lax.fori_loop(
                0, EMB_DIM, d_step, (z, z, z, z, z, z))
            acc = acc + _softplus(-jnp.clip(s, -10.0, 10.0))
            for nk in (n0, n1, n2, n3, n4):
                acc = acc + _softplus(jnp.clip(nk, -10.0, 10.0))
            return acc

        return lax.fori_loop(0, GROUPS, group_step, acc)

    acc = acc0
    pending = fire(0, 0)
    for c in range(n_chunks):
        nxt = fire(c + 1, (c + 1) % 2) if c + 1 < n_chunks else None
        for cp in pending:
            cp.wait()
        acc = compute(c, c % 2, acc)
        pending = nxt

    acc_buf[...] = acc
    pltpu.sync_copy(acc_buf, out.at[wid])


@jax.jit
def _sc_skipgram(pos_u, pos_v, neg_f, u_emb2, v_emb2):
    mesh = plsc.VectorSubcoreMesh(core_axis_name="c", subcore_axis_name="s")
    kcall = pl.kernel(
        _body,
        out_type=jax.ShapeDtypeStruct((NW, LANES), jnp.float32),
        mesh=mesh,
        compiler_params=pltpu.CompilerParams(needs_layout_passes=False),
        scratch_types=[
            pltpu.VMEM((PER_TILE,), jnp.int32),
            pltpu.VMEM((PER_TILE,), jnp.int32),
            pltpu.VMEM((PER_TILE * NUM_NEG,), jnp.int32),
            pltpu.VMEM((PER_TILE,), jnp.int32),
            pltpu.VMEM((PER_TILE,), jnp.int32),
            pltpu.VMEM((PER_TILE * NUM_NEG,), jnp.int32),
            pltpu.VMEM((CHUNK, 2 * EMB_DIM), jnp.float32),
            pltpu.VMEM((CHUNK, 2 * EMB_DIM), jnp.float32),
            pltpu.VMEM((CHUNK, 2 * EMB_DIM), jnp.float32),
            pltpu.VMEM((CHUNK, 2 * EMB_DIM), jnp.float32),
            pltpu.VMEM((NROWS, 2 * EMB_DIM), jnp.float32),
            pltpu.VMEM((NROWS, 2 * EMB_DIM), jnp.float32),
            pltpu.VMEM((LANES,), jnp.float32),
            pltpu.SemaphoreType.DMA,
            pltpu.SemaphoreType.DMA,
        ],
    )
    return kcall(pos_u, pos_v, neg_f, u_emb2, v_emb2)


def kernel(pos_u, pos_v, neg_v, u_emb, v_emb):
    batch = pos_u.shape[0]
    neg_f = neg_v.astype(jnp.int32).reshape(-1)
    u2 = u_emb.reshape(u_emb.shape[0] // 2, 2 * EMB_DIM)
    v2 = v_emb.reshape(v_emb.shape[0] // 2, 2 * EMB_DIM)
    partials = _sc_skipgram(pos_u.astype(jnp.int32), pos_v.astype(jnp.int32),
                            neg_f, u2, v2)
    return jnp.sum(partials) * (1.0 / batch)


# speedup vs baseline: 2.6674x; 1.6770x over previous
import jax
import jax.numpy as jnp
from jax import lax
from jax.experimental import pallas as pl
from jax.experimental.pallas import tpu as pltpu
from jax.experimental.pallas import tpu_sc as plsc

EMB_DIM = 64
NUM_NEG = 5
NC = 2
NS = 16
NW = NC * NS
LANES = 16
PER_TILE = 512
CHUNK = 64
GROUPS = CHUNK // LANES
NROWS = CHUNK * NUM_NEG

_LN2 = 0.6931471805599453


def _softplus(x):
    t = jnp.exp(x)
    y = 1.0 + t
    b = lax.bitcast_convert_type(y, jnp.int32)
    e = (b >> 23) - 127
    m = lax.bitcast_convert_type((b & 0x007FFFFF) | 0x3F800000, jnp.float32)
    z = (m - 1.0) / (m + 1.0)
    z2 = z * z
    p = z * (2.0 + z2 * (0.66666667 + z2 * (0.4 + z2 * (0.28571429 + z2 * 0.22222222))))
    return e.astype(jnp.float32) * _LN2 + p


def _dot4(a_ref, arow, b_ref, brow):
    p = a_ref[arow, pl.ds(0, LANES)] * b_ref[brow, pl.ds(0, LANES)]
    for k in range(1, 4):
        p = p + a_ref[arow, pl.ds(k * LANES, LANES)] * b_ref[brow, pl.ds(k * LANES, LANES)]
    return p


def _body(pos_u, pos_v, neg_f, u_emb, v_emb, out,
          pu_idx, pv_idx, ng_idx,
          u_buf0, u_buf1, v_buf0, v_buf1, n_buf0, n_buf1,
          acc_buf, sem0, sem1):
    wid = lax.axis_index("s") * NC + lax.axis_index("c")
    base = wid * PER_TILE

    pltpu.sync_copy(pos_u.at[pl.ds(base, PER_TILE)], pu_idx)
    pltpu.sync_copy(pos_v.at[pl.ds(base, PER_TILE)], pv_idx)
    pltpu.sync_copy(neg_f.at[pl.ds(base * NUM_NEG, PER_TILE * NUM_NEG)], ng_idx)

    u_bufs = (u_buf0, u_buf1)
    v_bufs = (v_buf0, v_buf1)
    n_bufs = (n_buf0, n_buf1)
    sems = (sem0, sem1)

    def fire(c, slot):
        ub, vb, nb = u_bufs[slot], v_bufs[slot], n_bufs[slot]
        sem = sems[slot]

        def fire_uv(g, _):
            uvec = pu_idx[pl.ds(c * CHUNK + g * LANES, LANES)]
            vvec = pv_idx[pl.ds(c * CHUNK + g * LANES, LANES)]
            row0 = g * LANES
            for j in range(LANES):
                pltpu.async_copy(u_emb.at[uvec[j]], ub.at[row0 + j], sem)
                pltpu.async_copy(v_emb.at[vvec[j]], vb.at[row0 + j], sem)
            return 0

        lax.fori_loop(0, GROUPS, fire_uv, 0)

        def fire_n(k, _):
            nvec = ng_idx[pl.ds(c * NROWS + k * LANES, LANES)]
            row0 = k * LANES
            for j in range(LANES):
                pltpu.async_copy(v_emb.at[nvec[j]], nb.at[row0 + j], sem)
            return 0

        lax.fori_loop(0, NROWS // LANES, fire_n, 0)

    def drain(slot):
        pltpu.make_async_copy(u_emb.at[pl.ds(0, CHUNK)], u_bufs[slot], sems[slot]).wait()
        pltpu.make_async_copy(u_emb.at[pl.ds(0, CHUNK)], v_bufs[slot], sems[slot]).wait()
        pltpu.make_async_copy(u_emb.at[pl.ds(0, NROWS)], n_bufs[slot], sems[slot]).wait()

    lane_iota = lax.iota(jnp.int32, LANES)

    def compute(slot, acc):
        ub, vb, nb = u_bufs[slot], v_bufs[slot], n_bufs[slot]

        def group_step(g, acc):
            def item_step(j, carry):
                sv, n0, n1, n2, n3, n4 = carry
                i = g * LANES + j
                s = jnp.sum(_dot4(ub, i, vb, i))
                s = jnp.maximum(jnp.minimum(s, 10.0), -10.0)
                msk = lane_iota == j
                sv = jnp.where(msk, -s, sv)
                outs = []
                for n, cur in enumerate((n0, n1, n2, n3, n4)):
                    t = jnp.sum(_dot4(nb, i * NUM_NEG + n, ub, i))
                    t = jnp.maximum(jnp.minimum(t, 10.0), -10.0)
                    outs.append(jnp.where(msk, t, cur))
                return (sv, *outs)

            z = jnp.zeros((LANES,), jnp.float32)
            sv, n0, n1, n2, n3, n4 = lax.fori_loop(
                0, LANES, item_step, (z, z, z, z, z, z))
            for vec in (sv, n0, n1, n2, n3, n4):
                acc = acc + _softplus(vec)
            return acc

        return lax.fori_loop(0, GROUPS, group_step, acc)

    n_chunks = PER_TILE // CHUNK
    acc = jnp.zeros((LANES,), jnp.float32)
    fire(0, 0)
    for c in range(n_chunks):
        if c + 1 < n_chunks:
            fire(c + 1, (c + 1) % 2)
        drain(c % 2)
        acc = compute(c % 2, acc)

    acc_buf[...] = acc
    pltpu.sync_copy(acc_buf, out.at[wid])


@jax.jit
def _sc_skipgram(pos_u, pos_v, neg_f, u_emb, v_emb):
    mesh = plsc.VectorSubcoreMesh(core_axis_name="c", subcore_axis_name="s")
    kcall = pl.kernel(
        _body,
        out_type=jax.ShapeDtypeStruct((NW, LANES), jnp.float32),
        mesh=mesh,
        compiler_params=pltpu.CompilerParams(needs_layout_passes=False),
        scratch_types=[
            pltpu.VMEM((PER_TILE,), jnp.int32),
            pltpu.VMEM((PER_TILE,), jnp.int32),
            pltpu.VMEM((PER_TILE * NUM_NEG,), jnp.int32),
            pltpu.VMEM((CHUNK, EMB_DIM), jnp.float32),
            pltpu.VMEM((CHUNK, EMB_DIM), jnp.float32),
            pltpu.VMEM((CHUNK, EMB_DIM), jnp.float32),
            pltpu.VMEM((CHUNK, EMB_DIM), jnp.float32),
            pltpu.VMEM((NROWS, EMB_DIM), jnp.float32),
            pltpu.VMEM((NROWS, EMB_DIM), jnp.float32),
            pltpu.VMEM((LANES,), jnp.float32),
            pltpu.SemaphoreType.DMA,
            pltpu.SemaphoreType.DMA,
        ],
    )
    return kcall(pos_u, pos_v, neg_f, u_emb, v_emb)


def kernel(pos_u, pos_v, neg_v, u_emb, v_emb):
    batch = pos_u.shape[0]
    neg_f = neg_v.astype(jnp.int32).reshape(-1)
    partials = _sc_skipgram(pos_u.astype(jnp.int32), pos_v.astype(jnp.int32),
                            neg_f, u_emb, v_emb)
    return jnp.sum(partials) * (1.0 / batch)
